# Initial kernel scaffold; baseline (speedup 1.0000x reference)
import jax
import jax.numpy as jnp
from jax.experimental import pallas as pl


def kernel(target, context, negatives, target_table, context_table):
    raise NotImplementedError("write your pallas kernel here")


# speedup vs baseline: 1.5916x; 1.5916x over previous
import functools

import jax
import jax.numpy as jnp
from jax import lax
from jax.experimental import pallas as pl
from jax.experimental.pallas import tpu as pltpu
from jax.experimental.pallas import tpu_sc as plsc

_B = 4096
_NEG = 20
_D = 64
_NCOL = _NEG + 1

_NC = 2
_NS = 16
_NW = _NC * _NS
_PER_W = _B // _NW
_CHUNK_I = 32
_NCHUNK = _PER_W // _CHUNK_I
_ROWS = _CHUNK_I * _NCOL
_GROUPS_PER_CHUNK = _CHUNK_I // 16


def _sc_body(tidx_hbm, cnidx_hbm, ttab_hbm, ctab_hbm, dots_hbm,
             idx_t, idx_cn, t_rows, cn_a, cn_b, out_v,
             sem_t, sem_a, sem_b):
    wid = lax.axis_index("s") * _NC + lax.axis_index("c")
    base = wid * _PER_W

    pltpu.sync_copy(tidx_hbm.at[pl.ds(base, _PER_W)], idx_t)
    pltpu.sync_copy(cnidx_hbm.at[pl.ds(base * _NCOL, _PER_W * _NCOL)], idx_cn)

    t_cp = pltpu.async_copy(ttab_hbm.at[idx_t], t_rows, sem_t)

    bufs = (cn_a, cn_b)
    sems = (sem_a, sem_b)

    def gather_chunk(c):
        return pltpu.async_copy(
            ctab_hbm.at[idx_cn.at[pl.ds(c * _ROWS, _ROWS)]],
            bufs[c % 2], sems[c % 2])

    cur_cp = gather_chunk(0)
    t_cp.wait()

    lanes = lax.iota(jnp.int32, 16)

    for c in range(_NCHUNK):
        nxt_cp = gather_chunk(c + 1) if c + 1 < _NCHUNK else None
        cur_cp.wait()
        buf = bufs[c % 2]
        for g2 in range(_GROUPS_PER_CHUNK):
            g = _GROUPS_PER_CHUNK * c + g2
            t_row = lanes + g * 16
            cn_row0 = (lanes + g2 * 16) * _NCOL

            def dbody(d, accs, t_row=t_row, cn_row0=cn_row0, buf=buf):
                dcol = jnp.full((16,), d, dtype=jnp.int32)
                t_col = plsc.load_gather(t_rows, [t_row, dcol])
                return tuple(
                    accs[jj] + t_col * plsc.load_gather(
                        buf, [cn_row0 + jj, dcol])
                    for jj in range(_NCOL))

            accs = lax.fori_loop(
                0, _D, dbody,
                tuple(jnp.zeros((16,), jnp.float32) for _ in range(_NCOL)))
            for jj in range(_NCOL):
                out_v[jj, pl.ds(g * 16, 16)] = accs[jj]
        cur_cp = nxt_cp

    pltpu.sync_copy(out_v, dots_hbm.at[wid])


@functools.partial(jax.jit, static_argnames=())
def _sc_dots(target, cn_idx, target_table, context_table):
    mesh = plsc.VectorSubcoreMesh(core_axis_name="c", subcore_axis_name="s",
                                  num_cores=_NC, num_subcores=_NS)
    f = pl.kernel(
        _sc_body,
        out_type=jax.ShapeDtypeStruct((_NW, _NCOL, _PER_W), jnp.float32),
        mesh=mesh,
        scratch_types=[
            pltpu.VMEM((_PER_W,), jnp.int32),
            pltpu.VMEM((_PER_W * _NCOL,), jnp.int32),
            pltpu.VMEM((_PER_W, _D), jnp.float32),
            pltpu.VMEM((_ROWS, _D), jnp.float32),
            pltpu.VMEM((_ROWS, _D), jnp.float32),
            pltpu.VMEM((_NCOL, _PER_W), jnp.float32),
            pltpu.SemaphoreType.DMA,
            pltpu.SemaphoreType.DMA,
            pltpu.SemaphoreType.DMA,
        ],
        compiler_params=pltpu.CompilerParams(needs_layout_passes=False,
                                             use_tc_tiling_on_sc=False),
        name="w2v_ns_dots_sc",
    )
    return f(target, cn_idx, target_table, context_table)


def _tc_body(dots_ref, out_ref):
    x = dots_ref[...]
    rows = lax.broadcasted_iota(jnp.int32, x.shape, 0)
    z = jnp.where(rows % _NCOL == 0, -x, x)
    out_ref[0, 0] = jnp.sum(jnp.logaddexp(0.0, z)) * (1.0 / _B)


def _tc_loss(dots):
    return pl.pallas_call(
        _tc_body,
        out_shape=jax.ShapeDtypeStruct((1, 1), jnp.float32),
        out_specs=pl.BlockSpec(memory_space=pltpu.SMEM),
        name="w2v_ns_loss_tc",
    )(dots)[0, 0]


def kernel(target, context, negatives, target_table, context_table):
    cn_idx = jnp.concatenate(
        [context.reshape(_B, 1).astype(jnp.int32),
         negatives.astype(jnp.int32)], axis=1).reshape(-1)
    dots = _sc_dots(target.astype(jnp.int32), cn_idx,
                    target_table, context_table)
    return _tc_loss(dots.reshape(_NW * _NCOL, _PER_W))
